# Initial kernel scaffold; baseline (speedup 1.0000x reference)
import jax
import jax.numpy as jnp
from jax.experimental import pallas as pl


def kernel(x, edge_index, edge_attr, fc1_w, fc1_b, k1_w, k1_b, k2_w, k2_b, root, conv_b, fc2_w, fc2_b):
    raise NotImplementedError("write your pallas kernel here")


# speedup vs baseline: 6.4400x; 6.4400x over previous
import functools

import jax
import jax.numpy as jnp
from jax import lax
from jax.experimental import pallas as pl
from jax.experimental.pallas import tpu as pltpu
from jax.experimental.pallas import tpu_sc as plsc

N = 10000
E = 160000
W = 16
DEPTH = 6

NC = 2
NS = 16
NW = NC * NS
EW = E // NW
CH = 1000
KCH = EW // CH
SUB = 125
JSUB = CH // SUB
ZR = 624
REM = N - NS * ZR

_mesh = functools.partial(
    plsc.VectorSubcoreMesh,
    core_axis_name="c", subcore_axis_name="s", num_cores=NC, num_subcores=NS,
)
_sc_params = pltpu.CompilerParams(use_tc_tiling_on_sc=False)


def _zero_spmem_slice(buf, agg, s):
    def zrow(i, _):
        buf[i, :] = jnp.zeros((W,), jnp.float32)
        return 0
    lax.fori_loop(0, ZR, zrow, 0)
    pltpu.sync_copy(buf.at[pl.ds(0, ZR)], agg.at[pl.ds(s * ZR, ZR)])

    @pl.when(s == NS - 1)
    def _():
        pltpu.sync_copy(buf.at[pl.ds(0, REM)], agg.at[pl.ds(NS * ZR, REM)])


def _copy_out_slice(agg, out_hbm, c, s):
    pltpu.sync_copy(agg.at[pl.ds(s * ZR, ZR)],
                    out_hbm.at[pl.ds(c * N + s * ZR, ZR)])

    @pl.when(s == NS - 1)
    def _():
        pltpu.sync_copy(agg.at[pl.ds(NS * ZR, REM)],
                        out_hbm.at[pl.ds(c * N + NS * ZR, REM)])


def _sc_edge_body(src3, dst3, e8, a_tab, out_hbm,
                  srcv, dstv, e8v, rowsv, msgv, agg, sem):
    c = lax.axis_index("c")
    s = lax.axis_index("s")
    wid = s * NC + c

    _zero_spmem_slice(msgv, agg, s)
    plsc.subcore_barrier()

    def chunk(k, _):
        g = wid * KCH + k
        base = wid * EW + k * CH
        cp1 = pltpu.async_copy(src3.at[g], srcv, sem)
        cp2 = pltpu.async_copy(dst3.at[g], dstv, sem)
        cp3 = pltpu.async_copy(e8.at[pl.ds(base, CH)], e8v, sem)
        cp1.wait()
        cp2.wait()
        cp3.wait()
        gs = [
            pltpu.async_copy(
                a_tab.at[srcv.at[j]], rowsv.at[pl.ds(j * SUB, SUB)], sem)
            for j in range(JSUB)
        ]
        for gcp in gs:
            gcp.wait()

        def edge(j, _):
            cf = e8v[j, :]
            acc = rowsv[j, pl.ds(4 * W, W)]
            acc = acc + cf[0] * rowsv[j, pl.ds(0, W)]
            acc = acc + cf[1] * rowsv[j, pl.ds(W, W)]
            acc = acc + cf[2] * rowsv[j, pl.ds(2 * W, W)]
            acc = acc + cf[3] * rowsv[j, pl.ds(3 * W, W)]
            msgv[j, :] = acc
            return 0
        lax.fori_loop(0, CH, edge, 0)

        for j in range(JSUB):
            pltpu.sync_copy(
                msgv.at[pl.ds(j * SUB, SUB)], agg.at[dstv.at[j]], add=True)
        return 0
    lax.fori_loop(0, KCH, chunk, 0)

    plsc.subcore_barrier()
    _copy_out_slice(agg, out_hbm, c, s)


def _sc_edge(src3, dst3, e8, a_tab):
    return pl.kernel(
        _sc_edge_body,
        out_type=jax.ShapeDtypeStruct((NC * N, W), jnp.float32),
        mesh=_mesh(),
        compiler_params=_sc_params,
        scratch_types=[
            pltpu.VMEM((JSUB, SUB), jnp.int32),
            pltpu.VMEM((JSUB, SUB), jnp.int32),
            pltpu.VMEM((CH, W), jnp.float32),
            pltpu.VMEM((CH, 5 * W), jnp.float32),
            pltpu.VMEM((CH, W), jnp.float32),
            pltpu.VMEM_SHARED((N, W), jnp.float32),
            pltpu.SemaphoreType.DMA,
        ],
    )(src3, dst3, e8, a_tab)


def _sc_count_body(dst3, out_hbm, dstv, onesv, zbuf, agg, sem):
    c = lax.axis_index("c")
    s = lax.axis_index("s")
    wid = s * NC + c

    _zero_spmem_slice(zbuf, agg, s)

    def orow(i, _):
        onesv[i, :] = jnp.ones((W,), jnp.float32)
        return 0
    lax.fori_loop(0, SUB, orow, 0)
    plsc.subcore_barrier()

    def chunk(k, _):
        g = wid * KCH + k
        pltpu.sync_copy(dst3.at[g], dstv)
        for j in range(JSUB):
            pltpu.sync_copy(onesv, agg.at[dstv.at[j]], add=True)
        return 0
    lax.fori_loop(0, KCH, chunk, 0)

    plsc.subcore_barrier()
    _copy_out_slice(agg, out_hbm, c, s)


def _sc_count(dst3):
    return pl.kernel(
        _sc_count_body,
        out_type=jax.ShapeDtypeStruct((NC * N, W), jnp.float32),
        mesh=_mesh(),
        compiler_params=_sc_params,
        scratch_types=[
            pltpu.VMEM((JSUB, SUB), jnp.int32),
            pltpu.VMEM((SUB, W), jnp.float32),
            pltpu.VMEM((ZR, W), jnp.float32),
            pltpu.VMEM_SHARED((N, W), jnp.float32),
            pltpu.SemaphoreType.DMA,
        ],
    )(dst3)


_EB = 8000
_NB = 2000


def _full(shape):
    return pl.BlockSpec(shape, lambda i: tuple(0 for _ in shape))


def _edge_mlp_body(ea_ref, w_ref, b_ref, o_ref):
    e = jnp.dot(ea_ref[:], w_ref[:], preferred_element_type=jnp.float32)
    e = jnp.maximum(e + b_ref[:], 0.0)
    o_ref[:] = jnp.concatenate(
        [e, jnp.zeros((_EB, 12), jnp.float32)], axis=1)


def _edge_mlp(ea, k1wT, k1b):
    return pl.pallas_call(
        _edge_mlp_body,
        grid=(E // _EB,),
        in_specs=[
            pl.BlockSpec((_EB, 4), lambda i: (i, 0)),
            _full((4, 4)),
            _full((1, 4)),
        ],
        out_specs=pl.BlockSpec((_EB, W), lambda i: (i, 0)),
        out_shape=jax.ShapeDtypeStruct((E, W), jnp.float32),
    )(ea, k1wT, k1b)


def _node_pro_body(x_ref, cnt_ref, fw_ref, fb_ref, k2_ref,
                   h_ref, ic_ref, a_ref):
    h = x_ref[:] * fw_ref[:] + fb_ref[:]
    h_ref[:] = h
    ic_ref[:] = 1.0 / jnp.maximum(cnt_ref[0] + cnt_ref[1], 1.0)
    a_ref[:] = jnp.dot(h, k2_ref[:], preferred_element_type=jnp.float32)


def _node_pro(x, cnt2, fw, fb, k2big):
    return pl.pallas_call(
        _node_pro_body,
        grid=(N // _NB,),
        in_specs=[
            pl.BlockSpec((_NB, 1), lambda i: (i, 0)),
            pl.BlockSpec((2, _NB, W), lambda i: (0, i, 0)),
            _full((1, W)),
            _full((1, W)),
            _full((W, 5 * W)),
        ],
        out_specs=[
            pl.BlockSpec((_NB, W), lambda i: (i, 0)),
            pl.BlockSpec((_NB, W), lambda i: (i, 0)),
            pl.BlockSpec((_NB, 5 * W), lambda i: (i, 0)),
        ],
        out_shape=[
            jax.ShapeDtypeStruct((N, W), jnp.float32),
            jax.ShapeDtypeStruct((N, W), jnp.float32),
            jax.ShapeDtypeStruct((N, 5 * W), jnp.float32),
        ],
    )(x, cnt2, fw, fb, k2big)


def _dense_body(g_ref, h_ref, ic_ref, root_ref, cb_ref, k2_ref,
                hn_ref, a_ref):
    agg = (g_ref[0] + g_ref[1]) * ic_ref[:]
    hr = jnp.dot(h_ref[:], root_ref[:], preferred_element_type=jnp.float32)
    hn = jnp.maximum(agg + hr + cb_ref[:], 0.0)
    hn_ref[:] = hn
    a_ref[:] = jnp.dot(hn, k2_ref[:], preferred_element_type=jnp.float32)


def _dense(agg2, h, invc, root, cb, k2big):
    return pl.pallas_call(
        _dense_body,
        grid=(N // _NB,),
        in_specs=[
            pl.BlockSpec((2, _NB, W), lambda i: (0, i, 0)),
            pl.BlockSpec((_NB, W), lambda i: (i, 0)),
            pl.BlockSpec((_NB, W), lambda i: (i, 0)),
            _full((W, W)),
            _full((1, W)),
            _full((W, 5 * W)),
        ],
        out_specs=[
            pl.BlockSpec((_NB, W), lambda i: (i, 0)),
            pl.BlockSpec((_NB, 5 * W), lambda i: (i, 0)),
        ],
        out_shape=[
            jax.ShapeDtypeStruct((N, W), jnp.float32),
            jax.ShapeDtypeStruct((N, 5 * W), jnp.float32),
        ],
    )(agg2, h, invc, root, cb, k2big)


def _final_body(g_ref, h_ref, ic_ref, root_ref, cb_ref, f2_ref, f2b_ref,
                o_ref):
    agg = (g_ref[0] + g_ref[1]) * ic_ref[:]
    hr = jnp.dot(h_ref[:], root_ref[:], preferred_element_type=jnp.float32)
    hn = jnp.maximum(agg + hr + cb_ref[:], 0.0)
    o_ref[:] = jnp.dot(hn, f2_ref[:], preferred_element_type=jnp.float32) \
        + f2b_ref[:]


def _final(agg2, h, invc, root, cb, f2, f2b):
    return pl.pallas_call(
        _final_body,
        grid=(N // _NB,),
        in_specs=[
            pl.BlockSpec((2, _NB, W), lambda i: (0, i, 0)),
            pl.BlockSpec((_NB, W), lambda i: (i, 0)),
            pl.BlockSpec((_NB, W), lambda i: (i, 0)),
            _full((W, W)),
            _full((1, W)),
            _full((W, 1)),
            _full((1, 1)),
        ],
        out_specs=pl.BlockSpec((_NB, 1), lambda i: (i, 0)),
        out_shape=jax.ShapeDtypeStruct((N, 1), jnp.float32),
    )(agg2, h, invc, root, cb, f2, f2b)


def kernel(x, edge_index, edge_attr, fc1_w, fc1_b, k1_w, k1_b, k2_w, k2_b,
           root, conv_b, fc2_w, fc2_b):
    src = edge_index[0].astype(jnp.int32)
    dst = edge_index[1].astype(jnp.int32)
    src3 = src.reshape(NW * KCH, JSUB, SUB)
    dst3 = dst.reshape(NW * KCH, JSUB, SUB)
    k2big = jnp.concatenate(
        [k2_w[:, r].reshape(W, W) for r in range(4)] + [k2_b.reshape(W, W)],
        axis=1)
    k1wT = k1_w.T
    k1b = k1_b.reshape(1, 4)
    fw = fc1_w.reshape(1, W)
    fb = fc1_b.reshape(1, W)
    cb = conv_b.reshape(1, W)
    f2 = fc2_w.T
    f2b = fc2_b.reshape(1, 1)

    e8 = _edge_mlp(edge_attr, k1wT, k1b)
    cnt2 = _sc_count(dst3).reshape(NC, N, W)
    h, invc, a_tab = _node_pro(x, cnt2, fw, fb, k2big)
    for _ in range(DEPTH - 1):
        agg2 = _sc_edge(src3, dst3, e8, a_tab).reshape(NC, N, W)
        h, a_tab = _dense(agg2, h, invc, root, cb, k2big)
    agg2 = _sc_edge(src3, dst3, e8, a_tab).reshape(NC, N, W)
    return _final(agg2, h, invc, root, cb, f2, f2b)


# speedup vs baseline: 9.3301x; 1.4488x over previous
import functools

import jax
import jax.numpy as jnp
from jax import lax
from jax.experimental import pallas as pl
from jax.experimental.pallas import tpu as pltpu
from jax.experimental.pallas import tpu_sc as plsc

N = 10000
E = 160000
W = 16
DEPTH = 6

NC = 2
NS = 16
NW = NC * NS
EW = E // NW
CH = 250
KCH = EW // CH
SUB = 125
JSUB = CH // SUB
ZR = 624
ZH = 208
REM = N - NS * ZR

_mesh = functools.partial(
    plsc.VectorSubcoreMesh,
    core_axis_name="c", subcore_axis_name="s", num_cores=NC, num_subcores=NS,
)
_sc_params = pltpu.CompilerParams(use_tc_tiling_on_sc=False)


def _zero_spmem_slice(buf, agg, s):
    @plsc.parallel_loop(0, ZH, unroll=4)
    def _(i):
        buf[i, :] = jnp.zeros((W,), jnp.float32)
    for z in range(ZR // ZH):
        pltpu.sync_copy(buf.at[pl.ds(0, ZH)],
                        agg.at[pl.ds(s * ZR + z * ZH, ZH)])

    @pl.when(s == NS - 1)
    def _():
        pltpu.sync_copy(buf.at[pl.ds(0, REM)], agg.at[pl.ds(NS * ZR, REM)])


def _copy_out_slice(agg, out_hbm, c, s):
    pltpu.sync_copy(agg.at[pl.ds(s * ZR, ZR)],
                    out_hbm.at[pl.ds(c * N + s * ZR, ZR)])

    @pl.when(s == NS - 1)
    def _():
        pltpu.sync_copy(agg.at[pl.ds(NS * ZR, REM)],
                        out_hbm.at[pl.ds(c * N + NS * ZR, REM)])


def _sc_edge_body(src3, dst3, e8, a_tab, out_hbm,
                  srcv, dstv, e8v, rowsv, msgv, agg, sst, sg, ssc):
    c = lax.axis_index("c")
    s = lax.axis_index("s")
    wid = s * NC + c

    def stage(k):
        g = wid * KCH + k
        base = wid * EW + k * CH
        return (
            pltpu.async_copy(src3.at[g], srcv.at[k % 2], sst),
            pltpu.async_copy(dst3.at[g], dstv.at[k % 3], sst),
            pltpu.async_copy(e8.at[pl.ds(base, CH)], e8v.at[k % 2], sst),
        )

    def fire_gathers(k):
        p = k % 2
        return [
            pltpu.async_copy(a_tab.at[srcv.at[p, j]],
                             rowsv.at[p, pl.ds(j * SUB, SUB)], sg)
            for j in range(JSUB)
        ]

    def compute(k):
        p, q = k % 2, k % 3

        @plsc.parallel_loop(0, CH, unroll=4)
        def _(j):
            cf = e8v[p, j, :]
            acc = rowsv[p, j, pl.ds(4 * W, W)]
            acc = acc + cf[0] * rowsv[p, j, pl.ds(0, W)]
            acc = acc + cf[1] * rowsv[p, j, pl.ds(W, W)]
            acc = acc + cf[2] * rowsv[p, j, pl.ds(2 * W, W)]
            acc = acc + cf[3] * rowsv[p, j, pl.ds(3 * W, W)]
            msgv[q, j, :] = acc

    def fire_scatters(k):
        q = k % 3
        return [
            pltpu.async_copy(msgv.at[q, pl.ds(j * SUB, SUB)],
                             agg.at[dstv.at[q, j]], ssc, add=True)
            for j in range(JSUB)
        ]

    st0 = stage(0)
    _zero_spmem_slice(msgv.at[0], agg, s)
    plsc.subcore_barrier()
    for cp in st0:
        cp.wait()
    pending_g = {0: fire_gathers(0)}
    pending_sc = {}
    for k in range(KCH):
        if k - 2 in pending_sc:
            for cp in pending_sc.pop(k - 2):
                cp.wait()
        st = stage(k + 1) if k + 1 < KCH else ()
        for cp in pending_g.pop(k):
            cp.wait()
        if k + 1 < KCH:
            for cp in st:
                cp.wait()
            pending_g[k + 1] = fire_gathers(k + 1)
        compute(k)
        pending_sc[k] = fire_scatters(k)
    for kk in sorted(pending_sc):
        for cp in pending_sc[kk]:
            cp.wait()

    plsc.subcore_barrier()
    _copy_out_slice(agg, out_hbm, c, s)


def _sc_edge(src3, dst3, e8, a_tab):
    return pl.kernel(
        _sc_edge_body,
        out_type=jax.ShapeDtypeStruct((NC * N, W), jnp.float32),
        mesh=_mesh(),
        compiler_params=_sc_params,
        scratch_types=[
            pltpu.VMEM((2, JSUB, SUB), jnp.int32),
            pltpu.VMEM((3, JSUB, SUB), jnp.int32),
            pltpu.VMEM((2, CH, W), jnp.float32),
            pltpu.VMEM((2, CH, 5 * W), jnp.float32),
            pltpu.VMEM((3, CH, W), jnp.float32),
            pltpu.VMEM_SHARED((N, W), jnp.float32),
            pltpu.SemaphoreType.DMA,
            pltpu.SemaphoreType.DMA,
            pltpu.SemaphoreType.DMA,
        ],
    )(src3, dst3, e8, a_tab)


def _sc_count_body(dst3, out_hbm, dstv, onesv, zbuf, agg, sem):
    c = lax.axis_index("c")
    s = lax.axis_index("s")
    wid = s * NC + c

    _zero_spmem_slice(zbuf, agg, s)

    def orow(i, _):
        onesv[i, :] = jnp.ones((W,), jnp.float32)
        return 0
    lax.fori_loop(0, SUB, orow, 0)
    plsc.subcore_barrier()

    def chunk(k, _):
        g = wid * KCH + k
        pltpu.sync_copy(dst3.at[g], dstv)
        for j in range(JSUB):
            pltpu.sync_copy(onesv, agg.at[dstv.at[j]], add=True)
        return 0
    lax.fori_loop(0, KCH, chunk, 0)

    plsc.subcore_barrier()
    _copy_out_slice(agg, out_hbm, c, s)


def _sc_count(dst3):
    return pl.kernel(
        _sc_count_body,
        out_type=jax.ShapeDtypeStruct((NC * N, W), jnp.float32),
        mesh=_mesh(),
        compiler_params=_sc_params,
        scratch_types=[
            pltpu.VMEM((JSUB, SUB), jnp.int32),
            pltpu.VMEM((SUB, W), jnp.float32),
            pltpu.VMEM((ZR, W), jnp.float32),
            pltpu.VMEM_SHARED((N, W), jnp.float32),
            pltpu.SemaphoreType.DMA,
        ],
    )(dst3)


_EB = 8000
_NB = 2000


def _full(shape):
    return pl.BlockSpec(shape, lambda i: tuple(0 for _ in shape))


def _edge_mlp_body(ea_ref, w_ref, b_ref, o_ref):
    e = jnp.dot(ea_ref[:], w_ref[:], preferred_element_type=jnp.float32)
    e = jnp.maximum(e + b_ref[:], 0.0)
    o_ref[:] = jnp.concatenate(
        [e, jnp.zeros((_EB, 12), jnp.float32)], axis=1)


def _edge_mlp(ea, k1wT, k1b):
    return pl.pallas_call(
        _edge_mlp_body,
        grid=(E // _EB,),
        in_specs=[
            pl.BlockSpec((_EB, 4), lambda i: (i, 0)),
            _full((4, 4)),
            _full((1, 4)),
        ],
        out_specs=pl.BlockSpec((_EB, W), lambda i: (i, 0)),
        out_shape=jax.ShapeDtypeStruct((E, W), jnp.float32),
    )(ea, k1wT, k1b)


def _node_pro_body(x_ref, cnt_ref, fw_ref, fb_ref, k2_ref,
                   h_ref, ic_ref, a_ref):
    h = x_ref[:] * fw_ref[:] + fb_ref[:]
    h_ref[:] = h
    ic_ref[:] = 1.0 / jnp.maximum(cnt_ref[0] + cnt_ref[1], 1.0)
    a_ref[:] = jnp.dot(h, k2_ref[:], preferred_element_type=jnp.float32)


def _node_pro(x, cnt2, fw, fb, k2big):
    return pl.pallas_call(
        _node_pro_body,
        grid=(N // _NB,),
        in_specs=[
            pl.BlockSpec((_NB, 1), lambda i: (i, 0)),
            pl.BlockSpec((2, _NB, W), lambda i: (0, i, 0)),
            _full((1, W)),
            _full((1, W)),
            _full((W, 5 * W)),
        ],
        out_specs=[
            pl.BlockSpec((_NB, W), lambda i: (i, 0)),
            pl.BlockSpec((_NB, W), lambda i: (i, 0)),
            pl.BlockSpec((_NB, 5 * W), lambda i: (i, 0)),
        ],
        out_shape=[
            jax.ShapeDtypeStruct((N, W), jnp.float32),
            jax.ShapeDtypeStruct((N, W), jnp.float32),
            jax.ShapeDtypeStruct((N, 5 * W), jnp.float32),
        ],
    )(x, cnt2, fw, fb, k2big)


def _dense_body(g_ref, h_ref, ic_ref, root_ref, cb_ref, k2_ref,
                hn_ref, a_ref):
    agg = (g_ref[0] + g_ref[1]) * ic_ref[:]
    hr = jnp.dot(h_ref[:], root_ref[:], preferred_element_type=jnp.float32)
    hn = jnp.maximum(agg + hr + cb_ref[:], 0.0)
    hn_ref[:] = hn
    a_ref[:] = jnp.dot(hn, k2_ref[:], preferred_element_type=jnp.float32)


def _dense(agg2, h, invc, root, cb, k2big):
    return pl.pallas_call(
        _dense_body,
        grid=(N // _NB,),
        in_specs=[
            pl.BlockSpec((2, _NB, W), lambda i: (0, i, 0)),
            pl.BlockSpec((_NB, W), lambda i: (i, 0)),
            pl.BlockSpec((_NB, W), lambda i: (i, 0)),
            _full((W, W)),
            _full((1, W)),
            _full((W, 5 * W)),
        ],
        out_specs=[
            pl.BlockSpec((_NB, W), lambda i: (i, 0)),
            pl.BlockSpec((_NB, 5 * W), lambda i: (i, 0)),
        ],
        out_shape=[
            jax.ShapeDtypeStruct((N, W), jnp.float32),
            jax.ShapeDtypeStruct((N, 5 * W), jnp.float32),
        ],
    )(agg2, h, invc, root, cb, k2big)


def _final_body(g_ref, h_ref, ic_ref, root_ref, cb_ref, f2_ref, f2b_ref,
                o_ref):
    agg = (g_ref[0] + g_ref[1]) * ic_ref[:]
    hr = jnp.dot(h_ref[:], root_ref[:], preferred_element_type=jnp.float32)
    hn = jnp.maximum(agg + hr + cb_ref[:], 0.0)
    o_ref[:] = jnp.dot(hn, f2_ref[:], preferred_element_type=jnp.float32) \
        + f2b_ref[:]


def _final(agg2, h, invc, root, cb, f2, f2b):
    return pl.pallas_call(
        _final_body,
        grid=(N // _NB,),
        in_specs=[
            pl.BlockSpec((2, _NB, W), lambda i: (0, i, 0)),
            pl.BlockSpec((_NB, W), lambda i: (i, 0)),
            pl.BlockSpec((_NB, W), lambda i: (i, 0)),
            _full((W, W)),
            _full((1, W)),
            _full((W, 1)),
            _full((1, 1)),
        ],
        out_specs=pl.BlockSpec((_NB, 1), lambda i: (i, 0)),
        out_shape=jax.ShapeDtypeStruct((N, 1), jnp.float32),
    )(agg2, h, invc, root, cb, f2, f2b)


def kernel(x, edge_index, edge_attr, fc1_w, fc1_b, k1_w, k1_b, k2_w, k2_b,
           root, conv_b, fc2_w, fc2_b):
    src = edge_index[0].astype(jnp.int32)
    dst = edge_index[1].astype(jnp.int32)
    src3 = src.reshape(NW * KCH, JSUB, SUB)
    dst3 = dst.reshape(NW * KCH, JSUB, SUB)
    k2big = jnp.concatenate(
        [k2_w[:, r].reshape(W, W) for r in range(4)] + [k2_b.reshape(W, W)],
        axis=1)
    k1wT = k1_w.T
    k1b = k1_b.reshape(1, 4)
    fw = fc1_w.reshape(1, W)
    fb = fc1_b.reshape(1, W)
    cb = conv_b.reshape(1, W)
    f2 = fc2_w.T
    f2b = fc2_b.reshape(1, 1)

    e8 = _edge_mlp(edge_attr, k1wT, k1b)
    cnt2 = _sc_count(dst3).reshape(NC, N, W)
    h, invc, a_tab = _node_pro(x, cnt2, fw, fb, k2big)
    for _ in range(DEPTH - 1):
        agg2 = _sc_edge(src3, dst3, e8, a_tab).reshape(NC, N, W)
        h, a_tab = _dense(agg2, h, invc, root, cb, k2big)
    agg2 = _sc_edge(src3, dst3, e8, a_tab).reshape(NC, N, W)
    return _final(agg2, h, invc, root, cb, f2, f2b)
